# baseline (speedup 1.0000x reference)
import functools

import jax
import jax.numpy as jnp
from jax import lax
from jax.experimental import pallas as pl
from jax.experimental.pallas import tpu as pltpu
from jax.experimental.pallas import tpu_sc as plsc

N = 10000
E = 320000
D = 128
R = 3
NB = 2

NC = 2
NS = 16
L = 16

K = 96
CH = 209
EPW = CH * K
EPAD = NS * EPW
PADDST = 1 << 30

SELCAP = EPW + 1064
NBUF = 2
PACK = 16384
PADPACK = 16000

HALF = 5120
NPADC = 5248
ZPT = NPADC // NS
OPT = HALF // NS
JUNKL = HALF

BN = 1000


def _hs_body(wc_ref, x_ref, w_ref, o_ref):
    r = pl.program_id(0)
    w = wc_ref[r, 0] * w_ref[0] + wc_ref[r, 1] * w_ref[1]
    o_ref[...] = jnp.dot(x_ref[...], w,
                         preferred_element_type=jnp.float32).astype(jnp.bfloat16)


def _hs_transform(x, weight, w_comp):
    nblk = N // BN
    return pl.pallas_call(
        _hs_body,
        grid=(R, nblk),
        in_specs=[
            pl.BlockSpec(memory_space=pltpu.SMEM),
            pl.BlockSpec((BN, D), lambda r, n: (n, 0)),
            pl.BlockSpec((NB, D, D), lambda r, n: (0, 0, 0)),
        ],
        out_specs=pl.BlockSpec((BN, D), lambda r, n: (r * nblk + n, 0)),
        out_shape=jax.ShapeDtypeStruct((R * N, D), jnp.bfloat16),
    )(w_comp, x, weight)


def _pack_body(src_ref, et_ref, dst_ref, o_ref):
    o_ref[...] = (et_ref[...] * N + src_ref[...]) * PACK + dst_ref[...]


def _pack_transform(src_m, et_m, dst_m):
    return pl.pallas_call(
        _pack_body,
        out_shape=jax.ShapeDtypeStruct(src_m.shape, jnp.int32),
    )(src_m, et_m, dst_m)


def _sc_body(hs_h, pk_h, zero_h, out_h,
             pk_v, sel_p, gbuf_v, cidx_v, rows_bf, frows,
             acc, gsem0, gsem1, ssem0, ssem1):
    c = lax.axis_index("c")
    s = lax.axis_index("s")

    pltpu.sync_copy(pk_h.at[s], pk_v)

    pltpu.sync_copy(zero_h.at[pl.ds(s * ZPT, ZPT)], acc.at[pl.ds(s * ZPT, ZPT)])

    plsc.subcore_barrier()

    lo = c * HALF

    def cbody(j, n):
        for k in range(K // L):
            pvec = pk_v[j, pl.ds(k * L, L)]
            dvec = pvec & (PACK - 1)
            mask = (dvec >= lo) & (dvec < lo + HALF)
            plsc.store_compressed(sel_p.at[pl.ds(n, L)], pvec, mask=mask)
            n = n + plsc.all_reduce_population_count(mask)[0]
        return n

    n = lax.fori_loop(0, CH, cbody, jnp.int32(0))

    jvec = jnp.zeros((L,), jnp.int32) + (lo + JUNKL)

    def jbody(i, carry):
        sel_p[pl.ds(n + i * L, L)] = jvec
        return carry

    lax.fori_loop(0, 65, jbody, 0)

    nq = (n + (K - 1)) // K
    mq = jnp.maximum((nq + 3) // 4 * 4, 8)

    gsems = (gsem0, gsem1)
    ssems = (ssem0, ssem1)

    def unpack(j, ib):
        for k in range(K // L):
            pvec = sel_p[pl.ds(j * K + k * L, L)]
            gbuf_v[ib, pl.ds(k * L, L)] = lax.shift_right_logical(pvec, 14)
            cidx_v[ib, pl.ds(k * L, L)] = (pvec & (PACK - 1)) - lo

    def convert(rb):
        himask = jnp.int32(-65536)

        def crow(r, carry):
            for g in range(D // 32):
                w = plsc.bitcast(rows_bf[rb, r, pl.ds(g * 32, 32)], jnp.int32)
                flo = plsc.bitcast(w << 16, jnp.float32)
                fhi = plsc.bitcast(w & himask, jnp.float32)
                frows[rb, r, pl.ds(g * 32, L)] = flo
                frows[rb, r, pl.ds(g * 32 + L, L)] = fhi
            return carry

        lax.fori_loop(0, K, crow, 0)

    def g_desc(j, rb, ib):
        return pltpu.make_async_copy(
            hs_h.at[gbuf_v.at[ib]], rows_bf.at[rb], gsems[rb])

    def s_desc(j, rb, ib):
        return pltpu.make_async_copy(
            frows.at[rb], acc.at[cidx_v.at[ib]], ssems[rb])

    def step(j, rb, ib, first, last):
        g_desc(j, rb, ib).wait()
        if not last:
            unpack(j + 1, (ib + 1) % 4)
            g_desc(j + 1, 1 - rb, (ib + 1) % 4).start()
        if not first:
            s_desc(j - 2, rb, (ib + 2) % 4).wait()
        convert(rb)
        s_desc(j, rb, ib).start(add=True)

    unpack(0, 0)
    g_desc(0, 0, 0).start()
    for u in range(4):
        step(u, u % 2, u, u < 2, False)

    def steady(i, carry):
        j = 4 * i
        for u in range(4):
            step(j + u, u % 2, u, False, False)
        return carry

    lax.fori_loop(1, mq // 4 - 1, steady, 0)

    jt = mq - 4
    for u in range(4):
        step(jt + u, u % 2, u, False, u == 3)
    s_desc(mq - 2, 0, 2).wait()
    s_desc(mq - 1, 1, 3).wait()

    plsc.subcore_barrier()

    pltpu.sync_copy(acc.at[pl.ds(s * OPT, OPT)], out_h.at[c, pl.ds(s * OPT, OPT)])


@functools.lru_cache(maxsize=1)
def _sc_gather_scatter():
    return pl.kernel(
        _sc_body,
        out_type=jax.ShapeDtypeStruct((NC, HALF, D), jnp.float32),
        mesh=plsc.VectorSubcoreMesh(
            core_axis_name="c", subcore_axis_name="s",
            num_cores=NC, num_subcores=NS),
        scratch_types=[
            pltpu.VMEM((CH, K), jnp.int32),
            pltpu.VMEM((SELCAP,), jnp.int32),
            pltpu.VMEM((4, K), jnp.int32),
            pltpu.VMEM((4, K), jnp.int32),
            pltpu.VMEM((NBUF, K, D), jnp.bfloat16),
            pltpu.VMEM((NBUF, K, D), jnp.float32),
            pltpu.VMEM_SHARED((NPADC, D), jnp.float32),
            pltpu.SemaphoreType.DMA,
            pltpu.SemaphoreType.DMA,
            pltpu.SemaphoreType.DMA,
            pltpu.SemaphoreType.DMA,
        ],
        compiler_params=pltpu.CompilerParams(
            use_tc_tiling_on_sc=False, needs_layout_passes=False),
    )


def kernel(x, edge_index, etypes, weight, w_comp, h_bias):
    src = edge_index[0].astype(jnp.int32)
    dst = edge_index[1].astype(jnp.int32)
    et = etypes.astype(jnp.int32)

    packed = _pack_transform(src.reshape(2500, 128), et.reshape(2500, 128),
                             dst.reshape(2500, 128)).reshape(E)
    pad = EPAD - E
    pk_p = jnp.concatenate(
        [packed, jnp.full((pad,), PADPACK, jnp.int32)]).reshape(NS, CH, K)

    weight_p = weight.reshape(NB, D, 4, 2, 16).swapaxes(3, 4).reshape(NB, D, D)
    hs = _hs_transform(x, weight_p, w_comp)
    zero = jnp.zeros((NPADC, D), jnp.float32)
    parts = _sc_gather_scatter()(hs, pk_p, zero)
    return parts.reshape(NC * HALF, D)[:N] + h_bias


# speedup vs baseline: 1.1096x; 1.1096x over previous
import functools

import jax
import jax.numpy as jnp
from jax import lax
from jax.experimental import pallas as pl
from jax.experimental.pallas import tpu as pltpu
from jax.experimental.pallas import tpu_sc as plsc

N = 10000
E = 320000
D = 128
R = 3
NB = 2

NC = 2
NS = 16
L = 16

K = 128
CH = 160
EPW = CH * K
EPAD = NS * EPW
PADDST = 1 << 30

SELCAP = EPW + 1064
NBUF = 2
PACK = 16384
PADPACK = 16000

SPLIT = 5000
HALF = 5120
NPADC = 5248
ZPT = NPADC // NS
OPT = HALF // NS
JUNKL = HALF

BN = 1000


def _hs_body(wc_ref, x_ref, w_ref, o_ref):
    r = pl.program_id(0)
    w = wc_ref[r, 0] * w_ref[0] + wc_ref[r, 1] * w_ref[1]
    o_ref[...] = jnp.dot(x_ref[...], w, preferred_element_type=jnp.float32)


def _hs_transform(x, weight, w_comp):
    nblk = N // BN
    return pl.pallas_call(
        _hs_body,
        grid=(R, nblk),
        in_specs=[
            pl.BlockSpec(memory_space=pltpu.SMEM),
            pl.BlockSpec((BN, D), lambda r, n: (n, 0)),
            pl.BlockSpec((NB, D, D), lambda r, n: (0, 0, 0)),
        ],
        out_specs=pl.BlockSpec((BN, D), lambda r, n: (r * nblk + n, 0)),
        out_shape=jax.ShapeDtypeStruct((R * N, D), jnp.float32),
    )(w_comp, x, weight)


def _pack_body(src_ref, et_ref, dst_ref, o_ref):
    o_ref[...] = (et_ref[...] * N + src_ref[...]) * PACK + dst_ref[...]


def _pack_transform(src_m, et_m, dst_m):
    return pl.pallas_call(
        _pack_body,
        out_shape=jax.ShapeDtypeStruct(src_m.shape, jnp.int32),
    )(src_m, et_m, dst_m)


def _sc_body(hs_h, pk_h, zero_h, out_h,
             pk_v, sel_p, gbuf_v, cidx_v, rows_v,
             acc, gsem0, gsem1, ssem0, ssem1):
    c = lax.axis_index("c")
    s = lax.axis_index("s")

    pltpu.sync_copy(pk_h.at[s], pk_v)

    pltpu.sync_copy(zero_h.at[pl.ds(s * ZPT, ZPT)], acc.at[pl.ds(s * ZPT, ZPT)])

    plsc.subcore_barrier()

    lo = c * SPLIT

    def cbody(j, n):
        for k in range(K // L):
            pvec = pk_v[j, pl.ds(k * L, L)]
            dvec = pvec & (PACK - 1)
            mask = (dvec >= lo) & (dvec < lo + SPLIT)
            plsc.store_compressed(sel_p.at[pl.ds(n, L)], pvec, mask=mask)
            n = n + plsc.all_reduce_population_count(mask)[0]
        return n

    n = lax.fori_loop(0, CH, cbody, jnp.int32(0))

    jvec = jnp.zeros((L,), jnp.int32) + (lo + JUNKL)

    def jbody(i, carry):
        sel_p[pl.ds(n + i * L, L)] = jvec
        return carry

    lax.fori_loop(0, 65, jbody, 0)

    nq = (n + (K - 1)) // K
    mq = jnp.maximum((nq + 1) // 2 * 2, 4)

    gsems = (gsem0, gsem1)
    ssems = (ssem0, ssem1)

    def unpack(j, b):
        for k in range(K // L):
            pvec = sel_p[pl.ds(j * K + k * L, L)]
            gbuf_v[b, pl.ds(k * L, L)] = lax.shift_right_logical(pvec, 14)
            cidx_v[b, pl.ds(k * L, L)] = (pvec & (PACK - 1)) - lo

    def g_desc(j, b):
        return pltpu.make_async_copy(hs_h.at[gbuf_v.at[b]], rows_v.at[b], gsems[b])

    def s_desc(j, b):
        return pltpu.make_async_copy(rows_v.at[b], acc.at[cidx_v.at[b]], ssems[b])

    unpack(0, 0)
    g_desc(0, 0).start()
    g_desc(0, 0).wait()
    unpack(1, 1)
    g_desc(1, 1).start()
    s_desc(0, 0).start(add=True)

    def steady(i, carry):
        j1 = 2 * i + 1
        g_desc(j1, 1).wait()
        s_desc(j1 - 1, 0).wait()
        unpack(j1 + 1, 0)
        g_desc(j1 + 1, 0).start()
        s_desc(j1, 1).start(add=True)
        j2 = 2 * i + 2
        g_desc(j2, 0).wait()
        s_desc(j2 - 1, 1).wait()
        unpack(j2 + 1, 1)
        g_desc(j2 + 1, 1).start()
        s_desc(j2, 0).start(add=True)
        return carry

    lax.fori_loop(0, (mq - 2) // 2, steady, 0)

    g_desc(mq - 1, 1).wait()
    s_desc(mq - 1, 1).start(add=True)
    s_desc(mq - 2, 0).wait()
    s_desc(mq - 1, 1).wait()

    plsc.subcore_barrier()

    pltpu.sync_copy(acc.at[pl.ds(s * OPT, OPT)], out_h.at[c, pl.ds(s * OPT, OPT)])


@functools.lru_cache(maxsize=1)
def _sc_gather_scatter():
    return pl.kernel(
        _sc_body,
        out_type=jax.ShapeDtypeStruct((NC, HALF, D), jnp.float32),
        mesh=plsc.VectorSubcoreMesh(
            core_axis_name="c", subcore_axis_name="s",
            num_cores=NC, num_subcores=NS),
        scratch_types=[
            pltpu.VMEM((CH, K), jnp.int32),
            pltpu.VMEM((SELCAP,), jnp.int32),
            pltpu.VMEM((NBUF, K), jnp.int32),
            pltpu.VMEM((NBUF, K), jnp.int32),
            pltpu.VMEM((NBUF, K, D), jnp.float32),
            pltpu.VMEM_SHARED((NPADC, D), jnp.float32),
            pltpu.SemaphoreType.DMA,
            pltpu.SemaphoreType.DMA,
            pltpu.SemaphoreType.DMA,
            pltpu.SemaphoreType.DMA,
        ],
        compiler_params=pltpu.CompilerParams(
            use_tc_tiling_on_sc=False, needs_layout_passes=False),
    )


def kernel(x, edge_index, etypes, weight, w_comp, h_bias):
    src = edge_index[0].astype(jnp.int32)
    dst = edge_index[1].astype(jnp.int32)
    et = etypes.astype(jnp.int32)

    packed = _pack_transform(src.reshape(2500, K), et.reshape(2500, K),
                             dst.reshape(2500, K)).reshape(E)
    pad = EPAD - E
    pk_p = jnp.concatenate(
        [packed, jnp.full((pad,), PADPACK, jnp.int32)]).reshape(NS, CH, K)

    hs = _hs_transform(x, weight, w_comp)
    zero = jnp.zeros((NPADC, D), jnp.float32)
    parts = _sc_gather_scatter()(hs, pk_p, zero)
    return jnp.concatenate([parts[0, :SPLIT], parts[1, :SPLIT]], axis=0) + h_bias


# speedup vs baseline: 1.1206x; 1.0099x over previous
import functools

import jax
import jax.numpy as jnp
from jax import lax
from jax.experimental import pallas as pl
from jax.experimental.pallas import tpu as pltpu
from jax.experimental.pallas import tpu_sc as plsc

N = 10000
E = 320000
D = 128
R = 3
NB = 2

NC = 2
NS = 16
L = 16

K = 128
CH = 160
EPW = CH * K
EPAD = NS * EPW
PADDST = 1 << 30

SELCAP = EPW + 1064
NBUF = 2
PACK = 16384
PADPACK = 16000

HALF = 5120
NPADC = 5248
ZPT = NPADC // NS
OPT = HALF // NS
JUNKL = HALF

BN = 1000


def _hs_body(wc_ref, x_ref, w_ref, o_ref):
    r = pl.program_id(0)
    w = wc_ref[r, 0] * w_ref[0] + wc_ref[r, 1] * w_ref[1]
    o_ref[...] = jnp.dot(x_ref[...], w, preferred_element_type=jnp.float32)


def _hs_transform(x, weight, w_comp):
    nblk = N // BN
    return pl.pallas_call(
        _hs_body,
        grid=(R, nblk),
        in_specs=[
            pl.BlockSpec(memory_space=pltpu.SMEM),
            pl.BlockSpec((BN, D), lambda r, n: (n, 0)),
            pl.BlockSpec((NB, D, D), lambda r, n: (0, 0, 0)),
        ],
        out_specs=pl.BlockSpec((BN, D), lambda r, n: (r * nblk + n, 0)),
        out_shape=jax.ShapeDtypeStruct((R * N, D), jnp.float32),
    )(w_comp, x, weight)


def _pack_body(src_ref, et_ref, dst_ref, o_ref):
    o_ref[...] = (et_ref[...] * N + src_ref[...]) * PACK + dst_ref[...]


def _pack_transform(src_m, et_m, dst_m):
    return pl.pallas_call(
        _pack_body,
        out_shape=jax.ShapeDtypeStruct(src_m.shape, jnp.int32),
    )(src_m, et_m, dst_m)


def _sc_body(hs_h, pk_h, zero_h, out_h,
             pk_v, sel_p, gbuf_v, cidx_v, rows_v,
             acc, gsem0, gsem1, ssem0, ssem1):
    c = lax.axis_index("c")
    s = lax.axis_index("s")

    pltpu.sync_copy(pk_h.at[s], pk_v)

    pltpu.sync_copy(zero_h.at[pl.ds(s * ZPT, ZPT)], acc.at[pl.ds(s * ZPT, ZPT)])

    plsc.subcore_barrier()

    lo = c * HALF

    def cbody(j, n):
        for k in range(K // L):
            pvec = pk_v[j, pl.ds(k * L, L)]
            dvec = pvec & (PACK - 1)
            mask = (dvec >= lo) & (dvec < lo + HALF)
            plsc.store_compressed(sel_p.at[pl.ds(n, L)], pvec, mask=mask)
            n = n + plsc.all_reduce_population_count(mask)[0]
        return n

    n = lax.fori_loop(0, CH, cbody, jnp.int32(0))

    jvec = jnp.zeros((L,), jnp.int32) + (lo + JUNKL)

    def jbody(i, carry):
        sel_p[pl.ds(n + i * L, L)] = jvec
        return carry

    lax.fori_loop(0, 65, jbody, 0)

    nq = (n + (K - 1)) // K
    mq = jnp.maximum((nq + 1) // 2 * 2, 4)

    gsems = (gsem0, gsem1)
    ssems = (ssem0, ssem1)

    def unpack(j, b):
        for k in range(K // L):
            pvec = sel_p[pl.ds(j * K + k * L, L)]
            gbuf_v[b, pl.ds(k * L, L)] = lax.shift_right_logical(pvec, 14)
            cidx_v[b, pl.ds(k * L, L)] = (pvec & (PACK - 1)) - lo

    def g_desc(j, b):
        return pltpu.make_async_copy(hs_h.at[gbuf_v.at[b]], rows_v.at[b], gsems[b])

    def s_desc(j, b):
        return pltpu.make_async_copy(rows_v.at[b], acc.at[cidx_v.at[b]], ssems[b])

    unpack(0, 0)
    g_desc(0, 0).start()
    g_desc(0, 0).wait()
    unpack(1, 1)
    g_desc(1, 1).start()
    s_desc(0, 0).start(add=True)

    def steady(i, carry):
        j1 = 2 * i + 1
        g_desc(j1, 1).wait()
        s_desc(j1 - 1, 0).wait()
        unpack(j1 + 1, 0)
        g_desc(j1 + 1, 0).start()
        s_desc(j1, 1).start(add=True)
        j2 = 2 * i + 2
        g_desc(j2, 0).wait()
        s_desc(j2 - 1, 1).wait()
        unpack(j2 + 1, 1)
        g_desc(j2 + 1, 1).start()
        s_desc(j2, 0).start(add=True)
        return carry

    lax.fori_loop(0, (mq - 2) // 2, steady, 0)

    g_desc(mq - 1, 1).wait()
    s_desc(mq - 1, 1).start(add=True)
    s_desc(mq - 2, 0).wait()
    s_desc(mq - 1, 1).wait()

    plsc.subcore_barrier()

    pltpu.sync_copy(acc.at[pl.ds(s * OPT, OPT)], out_h.at[c, pl.ds(s * OPT, OPT)])


@functools.lru_cache(maxsize=1)
def _sc_gather_scatter():
    return pl.kernel(
        _sc_body,
        out_type=jax.ShapeDtypeStruct((NC, HALF, D), jnp.float32),
        mesh=plsc.VectorSubcoreMesh(
            core_axis_name="c", subcore_axis_name="s",
            num_cores=NC, num_subcores=NS),
        scratch_types=[
            pltpu.VMEM((CH, K), jnp.int32),
            pltpu.VMEM((SELCAP,), jnp.int32),
            pltpu.VMEM((NBUF, K), jnp.int32),
            pltpu.VMEM((NBUF, K), jnp.int32),
            pltpu.VMEM((NBUF, K, D), jnp.float32),
            pltpu.VMEM_SHARED((NPADC, D), jnp.float32),
            pltpu.SemaphoreType.DMA,
            pltpu.SemaphoreType.DMA,
            pltpu.SemaphoreType.DMA,
            pltpu.SemaphoreType.DMA,
        ],
        compiler_params=pltpu.CompilerParams(
            use_tc_tiling_on_sc=False, needs_layout_passes=False),
    )


def kernel(x, edge_index, etypes, weight, w_comp, h_bias):
    src = edge_index[0].astype(jnp.int32)
    dst = edge_index[1].astype(jnp.int32)
    et = etypes.astype(jnp.int32)

    packed = _pack_transform(src.reshape(2500, K), et.reshape(2500, K),
                             dst.reshape(2500, K)).reshape(E)
    pad = EPAD - E
    pk_p = jnp.concatenate(
        [packed, jnp.full((pad,), PADPACK, jnp.int32)]).reshape(NS, CH, K)

    hs = _hs_transform(x, weight, w_comp)
    zero = jnp.zeros((NPADC, D), jnp.float32)
    parts = _sc_gather_scatter()(hs, pk_p, zero)
    return parts.reshape(NC * HALF, D)[:N] + h_bias
